# baseline (speedup 1.0000x reference)
import functools

import jax
import jax.numpy as jnp
from jax.experimental import pallas as pl
from jax.experimental.pallas import tpu as pltpu
from jax.experimental.pallas import tpu_sc as plsc

_H = 512
_W = 512
_BAND = 16
_CB = 512
_K = 16
_T = 21.0
_DUP_B = 3

_NB = _H // _BAND
_NC = _W // _CB
_NT = _NB * _NC
_N = 2048
_NE = _N * _DUP_B * (1 if _NC == 1 else 2)
_NSLOT = _NE + _NT * _K
_NROWS = _NSLOT + 8


def _raster_kernel(starts_ref, ncks_ref, counts_ref, params_ref, out_ref):
    b = pl.program_id(0)
    cb = pl.program_id(1)
    t = b * _NC + cb
    start = starts_ref[t]
    nck = ncks_ref[t]
    count = counts_ref[t]
    xs = (jax.lax.broadcasted_iota(jnp.int32, (1, _CB), 1)
          + cb * _CB).astype(jnp.float32) + 0.5
    y0 = (b * _BAND).astype(jnp.float32)
    kio = jax.lax.broadcasted_iota(jnp.int32, (_K, 1), 0)
    out_ref[...] = jnp.zeros_like(out_ref)

    def body(i, carry):
        off = start + i * _K
        p = params_ref[pl.ds(off, _K), :]
        inb = (kio + i * _K) < (count - 0)
        cx = p[:, 0:1]
        cy = p[:, 1:2]
        c0 = p[:, 2:3]
        c1 = p[:, 3:4]
        c2 = p[:, 4:5]
        w = jnp.where(inb, p[:, 5:8], 0.0)
        dx = xs - cx
        a = (0.5 * c0) * dx * dx
        c1dx = c1 * dx
        for y in range(_BAND):
            dy = (y0 + (y + 0.5)) - cy
            sig = a + (0.5 * c2) * (dy * dy) + dy * c1dx
            alpha = jnp.where(inb, jnp.exp(-sig), 0.0)
            contrib = jax.lax.dot_general(
                w, alpha, (((0,), (0,)), ((), ())),
                preferred_element_type=jnp.float32)
            out_ref[y, :, :] += contrib
        return carry

    jax.lax.fori_loop(0, nck, body, 0)


def _sc_bin(pos, gid, p):
    info = plsc.get_sparse_core_info()
    nw = info.num_cores * info.num_subcores
    bpw = _NE // nw

    mesh = plsc.VectorSubcoreMesh(core_axis_name="c", subcore_axis_name="s")

    @functools.partial(
        pl.kernel, mesh=mesh,
        out_type=jax.ShapeDtypeStruct((_NROWS, 128), jnp.float32),
        scratch_types=[
            pltpu.VMEM((bpw,), jnp.int32),
            pltpu.VMEM((bpw,), jnp.int32),
            pltpu.VMEM((bpw, 128), jnp.float32),
            pltpu.SemaphoreType.DMA,
        ],
    )
    def k(pos_hbm, gid_hbm, p_hbm, out_hbm, pos_v, idx_v, rows_v, sem):
        wid = jax.lax.axis_index("s") * info.num_cores + jax.lax.axis_index("c")
        base = wid * bpw
        pltpu.sync_copy(gid_hbm.at[pl.ds(base, bpw)], idx_v)
        pltpu.sync_copy(pos_hbm.at[pl.ds(base, bpw)], pos_v)
        pltpu.async_copy(p_hbm.at[idx_v], rows_v, sem).wait()
        pltpu.async_copy(rows_v, out_hbm.at[pos_v], sem).wait()

    return k(pos, gid, p)


def kernel(embed):
    e = embed.reshape(-1, 9).astype(jnp.float32)
    n = e.shape[0]
    xy = jnp.tanh(e[:, :2])
    cx = 0.5 * _W * (xy[:, 0] + 1.0)
    cy = 0.5 * _H * (xy[:, 1] + 1.0)
    l0 = e[:, 5] + 0.5
    l1 = e[:, 6]
    l2 = e[:, 7] + 0.5
    cov00 = l0 * l0
    cov01 = l0 * l1
    cov11 = l1 * l1 + l2 * l2
    det = cov00 * cov11 - cov01 * cov01
    conic0 = cov11 / det
    conic1 = -cov01 / det
    conic2 = cov00 / det
    w = e[:, 2:5] * jax.nn.sigmoid(e[:, 8:9])

    half_tr = 0.5 * (cov00 + cov11)
    lmax = half_tr + jnp.sqrt((0.5 * (cov00 - cov11)) ** 2 + cov01 * cov01)
    r = jnp.sqrt(2.0 * _T * lmax)

    P = jnp.concatenate(
        [jnp.stack([cx, cy, conic0, conic1, conic2], axis=1), w], axis=1)

    blo = jnp.ceil((cy - r - (_BAND - 0.5)) / _BAND).astype(jnp.int32)
    bhi = jnp.floor((cy + r - 0.5) / _BAND).astype(jnp.int32)
    bb = blo[:, None] + jnp.arange(_DUP_B, dtype=jnp.int32)[None, :]
    valid = (bb <= bhi[:, None]) & (bb >= 0) & (bb < _NB)
    tid = bb.reshape(-1)
    valid = valid.reshape(-1)
    gid = jnp.broadcast_to(
        jnp.arange(n, dtype=jnp.int32)[:, None], (n, _DUP_B)).reshape(-1)

    onehot = ((tid[:, None] == jnp.arange(_NT, dtype=jnp.int32)[None, :])
              & valid[:, None]).astype(jnp.float32)
    incl = jnp.cumsum(onehot, axis=0)
    rank = jnp.sum(incl * onehot, axis=1) - 1.0
    counts = incl[-1].astype(jnp.int32)
    ncks = (counts + _K - 1) // _K
    poff = _K * jnp.concatenate(
        [jnp.zeros((1,), jnp.int32), jnp.cumsum(ncks)])[:_NT]
    tclip = jnp.clip(tid, 0, _NT - 1)
    pos = poff[tclip] + rank.astype(jnp.int32)
    pos = jnp.where(valid, pos, _NSLOT)

    P128 = jnp.pad(P, ((0, 0), (0, 120)))
    E2 = _sc_bin(pos, gid, P128)

    grid_spec = pltpu.PrefetchScalarGridSpec(
        num_scalar_prefetch=3,
        grid=(_NB, _NC),
        in_specs=[pl.BlockSpec((_NROWS, 128), lambda b, c, *_: (0, 0))],
        out_specs=pl.BlockSpec((_BAND, 3, _CB), lambda b, c, *_: (b, 0, c)),
    )
    out = pl.pallas_call(
        _raster_kernel,
        grid_spec=grid_spec,
        out_shape=jax.ShapeDtypeStruct((_H, 3, _W), jnp.float32),
    )(poff, ncks, counts, E2)
    return jnp.transpose(out, (1, 0, 2))[None]


# speedup vs baseline: 1.3863x; 1.3863x over previous
import jax
import jax.numpy as jnp
from jax.experimental import pallas as pl
from jax.experimental.pallas import tpu as pltpu

_H = 512
_W = 512
_BAND = 16
_CB = 512
_K = 16
_T = 21.0
_DUP_B = 3


def _raster_kernel(starts_ref, ncks_ref, params_ref, out_ref):
    b = pl.program_id(0)
    cb = pl.program_id(1)
    t = b * (_W // _CB) + cb
    start = starts_ref[t]
    nck = ncks_ref[t]
    xs = (jax.lax.broadcasted_iota(jnp.int32, (1, _CB), 1)
          + cb * _CB).astype(jnp.float32) + 0.5
    y0 = (b * _BAND).astype(jnp.float32)
    out_ref[...] = jnp.zeros_like(out_ref)

    def body(i, carry):
        off = start + i * _K
        p = params_ref[pl.ds(off, _K), :]
        cx = p[:, 0:1]
        cy = p[:, 1:2]
        c0 = p[:, 2:3]
        c1 = p[:, 3:4]
        c2 = p[:, 4:5]
        w = p[:, 5:8]
        dx = xs - cx
        a = (0.5 * c0) * dx * dx
        c1dx = c1 * dx
        for y in range(_BAND):
            dy = (y0 + (y + 0.5)) - cy
            sig = a + (0.5 * c2) * (dy * dy) + dy * c1dx
            alpha = jnp.exp(-sig)
            contrib = jax.lax.dot_general(
                w, alpha, (((0,), (0,)), ((), ())),
                preferred_element_type=jnp.float32)
            out_ref[y, :, :] += contrib
        return carry

    jax.lax.fori_loop(0, nck, body, 0)


def kernel(embed):
    e = embed.reshape(-1, 9).astype(jnp.float32)
    n = e.shape[0]
    xy = jnp.tanh(e[:, :2])
    cx = 0.5 * _W * (xy[:, 0] + 1.0)
    cy = 0.5 * _H * (xy[:, 1] + 1.0)
    l0 = e[:, 5] + 0.5
    l1 = e[:, 6]
    l2 = e[:, 7] + 0.5
    cov00 = l0 * l0
    cov01 = l0 * l1
    cov11 = l1 * l1 + l2 * l2
    det = cov00 * cov11 - cov01 * cov01
    conic0 = cov11 / det
    conic1 = -cov01 / det
    conic2 = cov00 / det
    w = e[:, 2:5] * jax.nn.sigmoid(e[:, 8:9])

    half_tr = 0.5 * (cov00 + cov11)
    lmax = half_tr + jnp.sqrt((0.5 * (cov00 - cov11)) ** 2 + cov01 * cov01)
    r = jnp.sqrt(2.0 * _T * lmax)

    P = jnp.concatenate(
        [jnp.stack([cx, cy, conic0, conic1, conic2], axis=1), w], axis=1)

    nb = _H // _BAND
    nc = _W // _CB
    nt = nb * nc
    dup_c = 1 if nc == 1 else 2

    blo = jnp.ceil((cy - r - (_BAND - 0.5)) / _BAND).astype(jnp.int32)
    bhi = jnp.floor((cy + r - 0.5) / _BAND).astype(jnp.int32)
    bb = blo[:, None] + jnp.arange(_DUP_B, dtype=jnp.int32)[None, :]
    bvalid = (bb <= bhi[:, None]) & (bb >= 0) & (bb < nb)
    if nc == 1:
        cc = jnp.zeros((n, 1), jnp.int32)
        cvalid = jnp.ones((n, 1), bool)
    else:
        clo = jnp.ceil((cx - r - (_CB - 0.5)) / _CB).astype(jnp.int32)
        chi = jnp.floor((cx + r - 0.5) / _CB).astype(jnp.int32)
        cc = clo[:, None] + jnp.arange(dup_c, dtype=jnp.int32)[None, :]
        cvalid = (cc <= chi[:, None]) & (cc >= 0) & (cc < nc)
    tid = (bb[:, :, None] * nc + cc[:, None, :]).reshape(-1)
    valid = (bvalid[:, :, None] & cvalid[:, None, :]).reshape(-1)
    ne = n * _DUP_B * dup_c
    gid = jnp.broadcast_to(
        jnp.arange(n, dtype=jnp.int32)[:, None, None],
        (n, _DUP_B, dup_c)).reshape(-1)

    onehot = ((tid[:, None] == jnp.arange(nt, dtype=jnp.int32)[None, :])
              & valid[:, None]).astype(jnp.float32)
    blk = 512
    nblk = ne // blk
    oh3 = onehot.reshape(nblk, blk, nt)
    tril = jnp.tril(jnp.ones((blk, blk), jnp.float32))
    within = jnp.einsum('ij,cjt->cit', tril, oh3,
                        preferred_element_type=jnp.float32)
    bsum = oh3.sum(axis=1)
    bpre = jnp.cumsum(bsum, axis=0) - bsum
    incl = (within + bpre[:, None, :]).reshape(ne, nt)
    rank = jnp.sum(incl * onehot, axis=1) - 1.0
    counts = bsum.sum(axis=0)
    ncks = jnp.ceil(counts / _K).astype(jnp.int32)
    poff = _K * jnp.concatenate(
        [jnp.zeros((1,), jnp.int32), jnp.cumsum(ncks)])[:nt]
    nslot = ne + nt * _K
    tclip = jnp.clip(tid, 0, nt - 1)
    pos = poff[tclip] + rank.astype(jnp.int32)
    pos = jnp.where(valid, pos, nslot)
    src = jnp.full((nslot,), n, jnp.int32).at[pos].set(gid, mode='drop')
    P_ext = jnp.concatenate([P, jnp.zeros((1, 8), jnp.float32)], axis=0)
    E2 = P_ext[src]

    grid_spec = pltpu.PrefetchScalarGridSpec(
        num_scalar_prefetch=2,
        grid=(nb, nc),
        in_specs=[pl.BlockSpec((nslot, 8), lambda b, c, *_: (0, 0))],
        out_specs=pl.BlockSpec((_BAND, 3, _CB), lambda b, c, *_: (b, 0, c)),
    )
    out = pl.pallas_call(
        _raster_kernel,
        grid_spec=grid_spec,
        out_shape=jax.ShapeDtypeStruct((_H, 3, _W), jnp.float32),
    )(poff, ncks, E2)
    return jnp.transpose(out, (1, 0, 2))[None]
